# Initial kernel scaffold; baseline (speedup 1.0000x reference)
import jax
import jax.numpy as jnp
from jax.experimental import pallas as pl


def kernel(out1, out2, anchor1, anchor2, ot_cost):
    raise NotImplementedError("write your pallas kernel here")


# speedup vs baseline: 3.3629x; 3.3629x over previous
import functools

import jax
import jax.numpy as jnp
from jax import lax
from jax.experimental import pallas as pl
from jax.experimental.pallas import tpu as pltpu
from jax.experimental.pallas import tpu_sc as plsc

_N = 10000
_D = 128
_NA = 1024
_K = 10
_MARGIN = 1.0
_NPAD = 10240
_ROWS = 10
_COLS = 1024
_NW = 32
_P = (_NA * _K) // _NW
_CH = 64
_BIGI = 2 ** 30


def _mine_kernel(a1_ref, a2_ref, bt1_ref, bt2_ref, r1_ref, r2_ref, eye_ref,
                 neg1_ref, neg2_ref, widx1_ref, widx2_ref, dm_ref):
    i = pl.program_id(0)
    a1i = a1_ref[i]
    a2i = a2_ref[i]
    a1row = r1_ref[0]
    a2row = r2_ref[0]
    eye = eye_ref[...]
    dn = (((1,), (1,)), ((), ()))
    a1col = lax.dot_general(eye, a1row, dn,
                            preferred_element_type=jnp.float32)
    a2col = lax.dot_general(eye, a2row, dn,
                            preferred_element_type=jnp.float32)
    dm = jnp.sum(jnp.abs(a1row - a2row)) + _MARGIN

    iota_r = lax.broadcasted_iota(jnp.int32, (_ROWS, _COLS), 0)
    iota_c = lax.broadcasted_iota(jnp.int32, (_ROWS, _COLS), 1)
    flat = iota_r * _COLS + iota_c

    def topk(bt_ref, acol):
        rows = []
        for c in range(_ROWS):
            chunk = bt_ref[:, c * _COLS:(c + 1) * _COLS]
            rows.append(jnp.sum(jnp.abs(chunk - acol), axis=0,
                                keepdims=True))
        d = jnp.concatenate(rows, axis=0)
        idxs = []
        for _ in range(_K):
            m = jnp.min(d)
            idx = jnp.min(jnp.where(d == m, flat, jnp.int32(_BIGI)))
            idxs.append(idx)
            d = jnp.where(flat == idx, jnp.float32(jnp.inf), d)
        return idxs

    neg1 = topk(bt2_ref, a1col)
    neg2 = topk(bt1_ref, a2col)

    lane = lax.broadcasted_iota(jnp.int32, (1, 16), 1)

    def pack(vals):
        acc = jnp.zeros((1, 16), jnp.int32)
        for t, v in enumerate(vals):
            acc = jnp.where(lane == t, v, acc)
        return acc

    neg1_ref[0] = pack(neg1)
    neg2_ref[0] = pack(neg2)
    widx1_ref[0] = pack([a1i * _N + v for v in neg1])
    widx2_ref[0] = pack([v * _N + a2i for v in neg2])
    dm_ref[0] = jnp.zeros((1, 16), jnp.float32) + dm


def _mine_grid_spec():
    full = lambda i, a1, a2: (0, 0)
    return pltpu.PrefetchScalarGridSpec(
        num_scalar_prefetch=2,
        grid=(_NA,),
        in_specs=[
            pl.BlockSpec((_D, _NPAD), full),
            pl.BlockSpec((_D, _NPAD), full),
            pl.BlockSpec((1, 1, _D), lambda i, a1, a2: (a1[i], 0, 0)),
            pl.BlockSpec((1, 1, _D), lambda i, a1, a2: (a2[i], 0, 0)),
            pl.BlockSpec((_D, _D), full),
        ],
        out_specs=[
            pl.BlockSpec((1, 1, 16), lambda i, a1, a2: (i, 0, 0))
            for _ in range(5)
        ],
    )


def _mine_out_shapes():
    i32 = jnp.int32
    return [
        jax.ShapeDtypeStruct((_NA, 1, 16), i32),
        jax.ShapeDtypeStruct((_NA, 1, 16), i32),
        jax.ShapeDtypeStruct((_NA, 1, 16), i32),
        jax.ShapeDtypeStruct((_NA, 1, 16), i32),
        jax.ShapeDtypeStruct((_NA, 1, 16), jnp.float32),
    ]


def _loss_kernel(o1_hbm, o2_hbm, otf_hbm, nidx1_hbm, nidx2_hbm,
                 widx1_hbm, widx2_hbm, aidx1_hbm, aidx2_hbm, dmp_hbm,
                 out_hbm,
                 idx_v, ae_v, ne_v, w_v, dmp_v, res_v, sem):
    c = lax.axis_index("c")
    s = lax.axis_index("s")
    wid = s * 2 + c
    base = wid * _P

    pltpu.sync_copy(dmp_hbm.at[pl.ds(base, _P)], dmp_v.at[pl.ds(0, _P)])

    total = jnp.float32(0.0)
    dirs = [
        (aidx1_hbm, nidx1_hbm, widx1_hbm, o1_hbm, o2_hbm),
        (aidx2_hbm, nidx2_hbm, widx2_hbm, o2_hbm, o1_hbm),
    ]
    for aidx_hbm, nidx_hbm, widx_hbm, bank_a, bank_n in dirs:
        for j in range(_P // _CH):
            off = base + j * _CH
            pltpu.sync_copy(aidx_hbm.at[pl.ds(off, _CH)], idx_v)
            pltpu.async_copy(bank_a.at[idx_v],
                             ae_v.at[pl.ds(j * _CH, _CH), :], sem).wait()
            pltpu.sync_copy(nidx_hbm.at[pl.ds(off, _CH)], idx_v)
            pltpu.async_copy(bank_n.at[idx_v],
                             ne_v.at[pl.ds(j * _CH, _CH), :], sem).wait()
            pltpu.sync_copy(widx_hbm.at[pl.ds(off, _CH)], idx_v)
            pltpu.async_copy(otf_hbm.at[idx_v],
                             w_v.at[pl.ds(j * _CH, _CH)], sem).wait()

        def body(p, acc):
            w = w_v[pl.ds(p, 16)][0]
            dm = dmp_v[pl.ds(p, 16)][0]
            s16 = jnp.zeros((16,), jnp.float32)
            for ch in range(_D // 16):
                a = ae_v[p, pl.ds(ch * 16, 16)]
                n = ne_v[p, pl.ds(ch * 16, 16)]
                s16 = s16 + jnp.abs(a - w * n)
            sdist = s16[0]
            for l in range(1, 16):
                sdist = sdist + s16[l]
            return acc + jnp.maximum(dm - sdist, 0.0)

        total = lax.fori_loop(0, _P, body, total)

    res_v[...] = jnp.where(lax.iota(jnp.int32, 16) == 0, total, 0.0)
    pltpu.sync_copy(res_v, out_hbm.at[wid])


def _make_loss_call():
    mesh = plsc.VectorSubcoreMesh(core_axis_name="c", subcore_axis_name="s")
    return functools.partial(
        pl.kernel,
        mesh=mesh,
        out_type=jax.ShapeDtypeStruct((_NW, 16), jnp.float32),
        scratch_types=[
            pltpu.VMEM((_CH,), jnp.int32),
            pltpu.VMEM((_P, _D), jnp.float32),
            pltpu.VMEM((_P, _D), jnp.float32),
            pltpu.VMEM((_P + 16,), jnp.float32),
            pltpu.VMEM((_P + 16,), jnp.float32),
            pltpu.VMEM((16,), jnp.float32),
            pltpu.SemaphoreType.DMA,
        ],
    )(_loss_kernel)


def kernel(out1, out2, anchor1, anchor2, ot_cost):
    a1 = anchor1.astype(jnp.int32)
    a2 = anchor2.astype(jnp.int32)
    pad = _NPAD - _N
    bt1 = jnp.pad(out1.T, ((0, 0), (0, pad)), constant_values=1e9)
    bt2 = jnp.pad(out2.T, ((0, 0), (0, pad)), constant_values=1e9)
    r1 = out1.reshape(_N, 1, _D)
    r2 = out2.reshape(_N, 1, _D)
    eye = jnp.eye(_D, dtype=jnp.float32)

    neg1o, neg2o, widx1o, widx2o, dmo = pl.pallas_call(
        _mine_kernel,
        grid_spec=_mine_grid_spec(),
        out_shape=_mine_out_shapes(),
    )(a1, a2, bt1, bt2, r1, r2, eye)

    nidx1 = neg1o[:, 0, :_K].reshape(-1)
    nidx2 = neg2o[:, 0, :_K].reshape(-1)
    widx1 = widx1o[:, 0, :_K].reshape(-1)
    widx2 = widx2o[:, 0, :_K].reshape(-1)
    dmp = dmo[:, 0, :_K].reshape(-1)
    aidx1 = jnp.repeat(a1, _K)
    aidx2 = jnp.repeat(a2, _K)
    otf = ot_cost.reshape(-1)

    partial = _make_loss_call()(out1, out2, otf, nidx1, nidx2,
                                widx1, widx2, aidx1, aidx2, dmp)
    return jnp.sum(partial) / (_NA * _K)


# speedup vs baseline: 3.4116x; 1.0145x over previous
import functools

import jax
import jax.numpy as jnp
from jax import lax
from jax.experimental import pallas as pl
from jax.experimental.pallas import tpu as pltpu
from jax.experimental.pallas import tpu_sc as plsc

_N = 10000
_D = 128
_NA = 1024
_K = 10
_MARGIN = 1.0
_NPAD = 10240
_ROWS = 10
_COLS = 1024
_NW = 32
_P = (_NA * _K) // _NW
_CH = 64
_BIGI = 2 ** 30


def _mine_kernel(a1_ref, a2_ref, bt1_ref, bt2_ref,
                 r1a_ref, r2a_ref, r1b_ref, r2b_ref, eye_ref,
                 neg1_ref, neg2_ref, widx1_ref, widx2_ref, dm_ref):
    i = pl.program_id(0)
    eye = eye_ref[...]
    dn = (((1,), (1,)), ((), ()))

    iota_r = lax.broadcasted_iota(jnp.int32, (_ROWS, _COLS), 0)
    iota_c = lax.broadcasted_iota(jnp.int32, (_ROWS, _COLS), 1)
    flat = iota_r * _COLS + iota_c
    lane = lax.broadcasted_iota(jnp.int32, (1, 16), 1)

    def dist(bt_ref, acol):
        rows = []
        for c in range(_ROWS):
            chunk = bt_ref[:, c * _COLS:(c + 1) * _COLS]
            rows.append(jnp.sum(jnp.abs(chunk - acol), axis=0,
                                keepdims=True))
        return jnp.concatenate(rows, axis=0)

    def topk(d):
        idxs = []
        for _ in range(_K):
            m = jnp.min(d)
            idx = jnp.min(jnp.where(d == m, flat, jnp.int32(_BIGI)))
            idxs.append(idx)
            d = jnp.where(flat == idx, jnp.float32(jnp.inf), d)
        return idxs

    def pack(vals):
        acc = jnp.zeros((1, 16), jnp.int32)
        for t, v in enumerate(vals):
            acc = jnp.where(lane == t, v, acc)
        return acc

    halves = []
    for half, (r1_ref, r2_ref) in enumerate(((r1a_ref, r2a_ref),
                                             (r1b_ref, r2b_ref))):
        a1i = a1_ref[2 * i + half]
        a2i = a2_ref[2 * i + half]
        a1row = r1_ref[0]
        a2row = r2_ref[0]
        a1col = lax.dot_general(eye, a1row, dn,
                                preferred_element_type=jnp.float32)
        a2col = lax.dot_general(eye, a2row, dn,
                                preferred_element_type=jnp.float32)
        dm = jnp.sum(jnp.abs(a1row - a2row)) + _MARGIN
        d1 = dist(bt2_ref, a1col)
        d2 = dist(bt1_ref, a2col)
        neg1 = topk(d1)
        neg2 = topk(d2)
        halves.append((
            pack(neg1),
            pack(neg2),
            pack([a1i * _N + v for v in neg1]),
            pack([v * _N + a2i for v in neg2]),
            jnp.zeros((1, 16), jnp.float32) + dm,
        ))

    for ref, idx in ((neg1_ref, 0), (neg2_ref, 1), (widx1_ref, 2),
                     (widx2_ref, 3), (dm_ref, 4)):
        ref[0] = jnp.concatenate([halves[0][idx], halves[1][idx]], axis=0)


def _mine_grid_spec():
    full = lambda i, a1, a2: (0, 0)
    return pltpu.PrefetchScalarGridSpec(
        num_scalar_prefetch=2,
        grid=(_NA // 2,),
        in_specs=[
            pl.BlockSpec((_D, _NPAD), full),
            pl.BlockSpec((_D, _NPAD), full),
            pl.BlockSpec((1, 1, _D), lambda i, a1, a2: (a1[2 * i], 0, 0)),
            pl.BlockSpec((1, 1, _D), lambda i, a1, a2: (a2[2 * i], 0, 0)),
            pl.BlockSpec((1, 1, _D), lambda i, a1, a2: (a1[2 * i + 1], 0, 0)),
            pl.BlockSpec((1, 1, _D), lambda i, a1, a2: (a2[2 * i + 1], 0, 0)),
            pl.BlockSpec((_D, _D), full),
        ],
        out_specs=[
            pl.BlockSpec((1, 2, 16), lambda i, a1, a2: (i, 0, 0))
            for _ in range(5)
        ],
    )


def _mine_out_shapes():
    i32 = jnp.int32
    return [
        jax.ShapeDtypeStruct((_NA // 2, 2, 16), i32),
        jax.ShapeDtypeStruct((_NA // 2, 2, 16), i32),
        jax.ShapeDtypeStruct((_NA // 2, 2, 16), i32),
        jax.ShapeDtypeStruct((_NA // 2, 2, 16), i32),
        jax.ShapeDtypeStruct((_NA // 2, 2, 16), jnp.float32),
    ]


def _loss_kernel(o1_hbm, o2_hbm, otf_hbm, nidx1_hbm, nidx2_hbm,
                 widx1_hbm, widx2_hbm, aidx1_hbm, aidx2_hbm, dmp_hbm,
                 out_hbm,
                 idx_v, ae_v, ne_v, w_v, dmp_v, res_v, sem):
    c = lax.axis_index("c")
    s = lax.axis_index("s")
    wid = s * 2 + c
    base = wid * _P

    pltpu.sync_copy(dmp_hbm.at[pl.ds(base, _P)], dmp_v.at[pl.ds(0, _P)])

    total = jnp.float32(0.0)
    dirs = [
        (aidx1_hbm, nidx1_hbm, widx1_hbm, o1_hbm, o2_hbm),
        (aidx2_hbm, nidx2_hbm, widx2_hbm, o2_hbm, o1_hbm),
    ]
    for aidx_hbm, nidx_hbm, widx_hbm, bank_a, bank_n in dirs:
        for j in range(_P // _CH):
            off = base + j * _CH
            pltpu.sync_copy(aidx_hbm.at[pl.ds(off, _CH)], idx_v)
            pltpu.async_copy(bank_a.at[idx_v],
                             ae_v.at[pl.ds(j * _CH, _CH), :], sem).wait()
            pltpu.sync_copy(nidx_hbm.at[pl.ds(off, _CH)], idx_v)
            pltpu.async_copy(bank_n.at[idx_v],
                             ne_v.at[pl.ds(j * _CH, _CH), :], sem).wait()
            pltpu.sync_copy(widx_hbm.at[pl.ds(off, _CH)], idx_v)
            pltpu.async_copy(otf_hbm.at[idx_v],
                             w_v.at[pl.ds(j * _CH, _CH)], sem).wait()

        def body(p, acc):
            w = w_v[pl.ds(p, 16)][0]
            dm = dmp_v[pl.ds(p, 16)][0]
            s16 = jnp.zeros((16,), jnp.float32)
            for ch in range(_D // 16):
                a = ae_v[p, pl.ds(ch * 16, 16)]
                n = ne_v[p, pl.ds(ch * 16, 16)]
                s16 = s16 + jnp.abs(a - w * n)
            sdist = s16[0]
            for l in range(1, 16):
                sdist = sdist + s16[l]
            return acc + jnp.maximum(dm - sdist, 0.0)

        total = lax.fori_loop(0, _P, body, total)

    res_v[...] = jnp.where(lax.iota(jnp.int32, 16) == 0, total, 0.0)
    pltpu.sync_copy(res_v, out_hbm.at[wid])


def _make_loss_call():
    mesh = plsc.VectorSubcoreMesh(core_axis_name="c", subcore_axis_name="s")
    return functools.partial(
        pl.kernel,
        mesh=mesh,
        out_type=jax.ShapeDtypeStruct((_NW, 16), jnp.float32),
        scratch_types=[
            pltpu.VMEM((_CH,), jnp.int32),
            pltpu.VMEM((_P, _D), jnp.float32),
            pltpu.VMEM((_P, _D), jnp.float32),
            pltpu.VMEM((_P + 16,), jnp.float32),
            pltpu.VMEM((_P + 16,), jnp.float32),
            pltpu.VMEM((16,), jnp.float32),
            pltpu.SemaphoreType.DMA,
        ],
    )(_loss_kernel)


def kernel(out1, out2, anchor1, anchor2, ot_cost):
    a1 = anchor1.astype(jnp.int32)
    a2 = anchor2.astype(jnp.int32)
    pad = _NPAD - _N
    bt1 = jnp.pad(out1.T, ((0, 0), (0, pad)), constant_values=1e9)
    bt2 = jnp.pad(out2.T, ((0, 0), (0, pad)), constant_values=1e9)
    r1 = out1.reshape(_N, 1, _D)
    r2 = out2.reshape(_N, 1, _D)
    eye = jnp.eye(_D, dtype=jnp.float32)

    neg1o, neg2o, widx1o, widx2o, dmo = pl.pallas_call(
        _mine_kernel,
        grid_spec=_mine_grid_spec(),
        out_shape=_mine_out_shapes(),
    )(a1, a2, bt1, bt2, r1, r2, r1, r2, eye)

    nidx1 = neg1o[:, :, :_K].reshape(-1)
    nidx2 = neg2o[:, :, :_K].reshape(-1)
    widx1 = widx1o[:, :, :_K].reshape(-1)
    widx2 = widx2o[:, :, :_K].reshape(-1)
    dmp = dmo[:, :, :_K].reshape(-1)
    aidx1 = jnp.repeat(a1, _K)
    aidx2 = jnp.repeat(a2, _K)
    otf = ot_cost.reshape(-1)

    partial = _make_loss_call()(out1, out2, otf, nidx1, nidx2,
                                widx1, widx2, aidx1, aidx2, dmp)
    return jnp.sum(partial) / (_NA * _K)


# speedup vs baseline: 3.6174x; 1.0603x over previous
import functools

import jax
import jax.numpy as jnp
from jax import lax
from jax.experimental import pallas as pl
from jax.experimental.pallas import tpu as pltpu
from jax.experimental.pallas import tpu_sc as plsc

_N = 10000
_D = 128
_NA = 1024
_K = 10
_MARGIN = 1.0
_NPAD = 10240
_ROWS = 10
_COLS = 1024
_NW = 32
_P = (_NA * _K) // _NW
_CH = 64
_BIGI = 2 ** 30


def _mine_kernel(a1_ref, a2_ref, bt1_ref, bt2_ref,
                 r1a_ref, r2a_ref, r1b_ref, r2b_ref, eye_ref,
                 neg1_ref, neg2_ref, widx1_ref, widx2_ref, dm_ref):
    i = pl.program_id(0)
    eye = eye_ref[...]
    dn = (((1,), (1,)), ((), ()))

    iota_r = lax.broadcasted_iota(jnp.int32, (_ROWS, _COLS), 0)
    iota_c = lax.broadcasted_iota(jnp.int32, (_ROWS, _COLS), 1)
    flat = iota_r * _COLS + iota_c
    lane = lax.broadcasted_iota(jnp.int32, (1, 16), 1)

    def dist2(bt_ref, acolb_x, acolb_y):
        rows_x, rows_y = [], []
        for c in range(_ROWS):
            accx = accy = None
            for s in range(_D // 8):
                strip = bt_ref[8 * s:8 * s + 8, c * _COLS:(c + 1) * _COLS]
                tx = jnp.abs(strip - acolb_x[8 * s:8 * s + 8, :])
                ty = jnp.abs(strip - acolb_y[8 * s:8 * s + 8, :])
                accx = tx if accx is None else accx + tx
                accy = ty if accy is None else accy + ty
            rows_x.append(jnp.sum(accx, axis=0, keepdims=True))
            rows_y.append(jnp.sum(accy, axis=0, keepdims=True))
        return (jnp.concatenate(rows_x, axis=0),
                jnp.concatenate(rows_y, axis=0))

    def topk(d):
        idxs = []
        for _ in range(_K):
            m = jnp.min(d)
            idx = jnp.min(jnp.where(d == m, flat, jnp.int32(_BIGI)))
            idxs.append(idx)
            d = jnp.where(flat == idx, jnp.float32(jnp.inf), d)
        return idxs

    def pack(vals):
        acc = jnp.zeros((1, 16), jnp.int32)
        for t, v in enumerate(vals):
            acc = jnp.where(lane == t, v, acc)
        return acc

    ones_row = jnp.ones((1, _COLS), jnp.float32)

    def colb(arow):
        acol = lax.dot_general(eye, arow, dn,
                               preferred_element_type=jnp.float32)
        return lax.dot_general(acol, ones_row, (((1,), (0,)), ((), ())),
                               preferred_element_type=jnp.float32)

    a1rows = (r1a_ref[0], r1b_ref[0])
    a2rows = (r2a_ref[0], r2b_ref[0])
    cb1 = (colb(a1rows[0]), colb(a1rows[1]))
    cb2 = (colb(a2rows[0]), colb(a2rows[1]))

    d1a, d1b = dist2(bt2_ref, cb1[0], cb1[1])
    d2a, d2b = dist2(bt1_ref, cb2[0], cb2[1])

    halves = []
    for half, (dd1, dd2) in enumerate(((d1a, d2a), (d1b, d2b))):
        a1i = a1_ref[2 * i + half]
        a2i = a2_ref[2 * i + half]
        dm = jnp.sum(jnp.abs(a1rows[half] - a2rows[half])) + _MARGIN
        neg1 = topk(dd1)
        neg2 = topk(dd2)
        halves.append((
            pack(neg1),
            pack(neg2),
            pack([a1i * _N + v for v in neg1]),
            pack([v * _N + a2i for v in neg2]),
            jnp.zeros((1, 16), jnp.float32) + dm,
        ))

    for ref, idx in ((neg1_ref, 0), (neg2_ref, 1), (widx1_ref, 2),
                     (widx2_ref, 3), (dm_ref, 4)):
        ref[0] = jnp.concatenate([halves[0][idx], halves[1][idx]], axis=0)


def _mine_grid_spec():
    full = lambda i, a1, a2: (0, 0)
    return pltpu.PrefetchScalarGridSpec(
        num_scalar_prefetch=2,
        grid=(_NA // 2,),
        in_specs=[
            pl.BlockSpec((_D, _NPAD), full),
            pl.BlockSpec((_D, _NPAD), full),
            pl.BlockSpec((1, 1, _D), lambda i, a1, a2: (a1[2 * i], 0, 0)),
            pl.BlockSpec((1, 1, _D), lambda i, a1, a2: (a2[2 * i], 0, 0)),
            pl.BlockSpec((1, 1, _D), lambda i, a1, a2: (a1[2 * i + 1], 0, 0)),
            pl.BlockSpec((1, 1, _D), lambda i, a1, a2: (a2[2 * i + 1], 0, 0)),
            pl.BlockSpec((_D, _D), full),
        ],
        out_specs=[
            pl.BlockSpec((1, 2, 16), lambda i, a1, a2: (i, 0, 0))
            for _ in range(5)
        ],
    )


def _mine_out_shapes():
    i32 = jnp.int32
    return [
        jax.ShapeDtypeStruct((_NA // 2, 2, 16), i32),
        jax.ShapeDtypeStruct((_NA // 2, 2, 16), i32),
        jax.ShapeDtypeStruct((_NA // 2, 2, 16), i32),
        jax.ShapeDtypeStruct((_NA // 2, 2, 16), i32),
        jax.ShapeDtypeStruct((_NA // 2, 2, 16), jnp.float32),
    ]


def _loss_kernel(o1_hbm, o2_hbm, otf_hbm, nidx1_hbm, nidx2_hbm,
                 widx1_hbm, widx2_hbm, aidx1_hbm, aidx2_hbm, dmp_hbm,
                 out_hbm,
                 idx_v, ae_v, ne_v, w_v, dmp_v, res_v, sem):
    c = lax.axis_index("c")
    s = lax.axis_index("s")
    wid = s * 2 + c
    base = wid * _P

    pltpu.sync_copy(dmp_hbm.at[pl.ds(base, _P)], dmp_v.at[pl.ds(0, _P)])

    total = jnp.float32(0.0)
    dirs = [
        (aidx1_hbm, nidx1_hbm, widx1_hbm, o1_hbm, o2_hbm),
        (aidx2_hbm, nidx2_hbm, widx2_hbm, o2_hbm, o1_hbm),
    ]
    for aidx_hbm, nidx_hbm, widx_hbm, bank_a, bank_n in dirs:
        for j in range(_P // _CH):
            off = base + j * _CH
            pltpu.sync_copy(aidx_hbm.at[pl.ds(off, _CH)], idx_v)
            pltpu.async_copy(bank_a.at[idx_v],
                             ae_v.at[pl.ds(j * _CH, _CH), :], sem).wait()
            pltpu.sync_copy(nidx_hbm.at[pl.ds(off, _CH)], idx_v)
            pltpu.async_copy(bank_n.at[idx_v],
                             ne_v.at[pl.ds(j * _CH, _CH), :], sem).wait()
            pltpu.sync_copy(widx_hbm.at[pl.ds(off, _CH)], idx_v)
            pltpu.async_copy(otf_hbm.at[idx_v],
                             w_v.at[pl.ds(j * _CH, _CH)], sem).wait()

        def body(p, acc):
            w = w_v[pl.ds(p, 16)][0]
            dm = dmp_v[pl.ds(p, 16)][0]
            s16 = jnp.zeros((16,), jnp.float32)
            for ch in range(_D // 16):
                a = ae_v[p, pl.ds(ch * 16, 16)]
                n = ne_v[p, pl.ds(ch * 16, 16)]
                s16 = s16 + jnp.abs(a - w * n)
            sdist = s16[0]
            for l in range(1, 16):
                sdist = sdist + s16[l]
            return acc + jnp.maximum(dm - sdist, 0.0)

        total = lax.fori_loop(0, _P, body, total)

    res_v[...] = jnp.where(lax.iota(jnp.int32, 16) == 0, total, 0.0)
    pltpu.sync_copy(res_v, out_hbm.at[wid])


def _make_loss_call():
    mesh = plsc.VectorSubcoreMesh(core_axis_name="c", subcore_axis_name="s")
    return functools.partial(
        pl.kernel,
        mesh=mesh,
        out_type=jax.ShapeDtypeStruct((_NW, 16), jnp.float32),
        scratch_types=[
            pltpu.VMEM((_CH,), jnp.int32),
            pltpu.VMEM((_P, _D), jnp.float32),
            pltpu.VMEM((_P, _D), jnp.float32),
            pltpu.VMEM((_P + 16,), jnp.float32),
            pltpu.VMEM((_P + 16,), jnp.float32),
            pltpu.VMEM((16,), jnp.float32),
            pltpu.SemaphoreType.DMA,
        ],
    )(_loss_kernel)


def kernel(out1, out2, anchor1, anchor2, ot_cost):
    a1 = anchor1.astype(jnp.int32)
    a2 = anchor2.astype(jnp.int32)
    pad = _NPAD - _N
    bt1 = jnp.pad(out1.T, ((0, 0), (0, pad)), constant_values=1e9)
    bt2 = jnp.pad(out2.T, ((0, 0), (0, pad)), constant_values=1e9)
    r1 = out1.reshape(_N, 1, _D)
    r2 = out2.reshape(_N, 1, _D)
    eye = jnp.eye(_D, dtype=jnp.float32)

    neg1o, neg2o, widx1o, widx2o, dmo = pl.pallas_call(
        _mine_kernel,
        grid_spec=_mine_grid_spec(),
        out_shape=_mine_out_shapes(),
    )(a1, a2, bt1, bt2, r1, r2, r1, r2, eye)

    nidx1 = neg1o[:, :, :_K].reshape(-1)
    nidx2 = neg2o[:, :, :_K].reshape(-1)
    widx1 = widx1o[:, :, :_K].reshape(-1)
    widx2 = widx2o[:, :, :_K].reshape(-1)
    dmp = dmo[:, :, :_K].reshape(-1)
    aidx1 = jnp.repeat(a1, _K)
    aidx2 = jnp.repeat(a2, _K)
    otf = ot_cost.reshape(-1)

    partial = _make_loss_call()(out1, out2, otf, nidx1, nidx2,
                                widx1, widx2, aidx1, aidx2, dmp)
    return jnp.sum(partial) / (_NA * _K)


# speedup vs baseline: 9.5996x; 2.6538x over previous
import functools

import jax
import jax.numpy as jnp
from jax import lax
from jax.experimental import pallas as pl
from jax.experimental.pallas import tpu as pltpu
from jax.experimental.pallas import tpu_sc as plsc

_N = 10000
_D = 128
_NA = 1024
_K = 10
_MARGIN = 1.0
_NPAD = 10240
_ROWS = 10
_COLS = 1024
_NW = 32
_P = (_NA * _K) // _NW
_CH = 64
_BIGI = 2 ** 30


def _mine_kernel(a1_ref, a2_ref, bt1_ref, bt2_ref,
                 r1a_ref, r2a_ref, r1b_ref, r2b_ref, eye_ref,
                 neg1_ref, neg2_ref, widx1_ref, widx2_ref, dm_ref):
    i = pl.program_id(0)
    eye = eye_ref[...]
    dn = (((1,), (1,)), ((), ()))

    iota_r = lax.broadcasted_iota(jnp.int32, (4, _ROWS, _COLS), 1)
    iota_c = lax.broadcasted_iota(jnp.int32, (4, _ROWS, _COLS), 2)
    flat3 = iota_r * _COLS + iota_c

    def dist2(bt_ref, acolb_x, acolb_y):
        rows_x, rows_y = [], []
        for c in range(_ROWS):
            accx = accy = None
            for s in range(_D // 8):
                strip = bt_ref[8 * s:8 * s + 8, c * _COLS:(c + 1) * _COLS]
                tx = jnp.abs(strip - acolb_x[8 * s:8 * s + 8, :])
                ty = jnp.abs(strip - acolb_y[8 * s:8 * s + 8, :])
                accx = tx if accx is None else accx + tx
                accy = ty if accy is None else accy + ty
            rows_x.append(jnp.sum(accx, axis=0, keepdims=True))
            rows_y.append(jnp.sum(accy, axis=0, keepdims=True))
        return (jnp.concatenate(rows_x, axis=0),
                jnp.concatenate(rows_y, axis=0))

    def colb(arow):
        acol = lax.dot_general(eye, arow, dn,
                               preferred_element_type=jnp.float32)
        return jnp.broadcast_to(acol, (_D, _COLS))

    a1rows = (r1a_ref[0], r1b_ref[0])
    a2rows = (r2a_ref[0], r2b_ref[0])

    d1a, d1b = dist2(bt2_ref, colb(a1rows[0]), colb(a1rows[1]))
    d2a, d2b = dist2(bt1_ref, colb(a2rows[0]), colb(a2rows[1]))

    d4 = jnp.stack([d1a, d1b, d2a, d2b], axis=0)
    idxs = []
    for _ in range(_K):
        m4 = jnp.min(d4, axis=(1, 2), keepdims=True)
        i4 = jnp.min(jnp.where(d4 == m4, flat3, jnp.int32(_BIGI)),
                     axis=(1, 2), keepdims=True)
        idxs.append(i4)
        d4 = jnp.where(flat3 == i4, jnp.float32(jnp.inf), d4)
    cat = jnp.concatenate(idxs, axis=2)

    a1i0 = a1_ref[2 * i]
    a1i1 = a1_ref[2 * i + 1]
    a2i0 = a2_ref[2 * i]
    a2i1 = a2_ref[2 * i + 1]
    dm0 = jnp.sum(jnp.abs(a1rows[0] - a2rows[0])) + _MARGIN
    dm1 = jnp.sum(jnp.abs(a1rows[1] - a2rows[1])) + _MARGIN

    zpad = jnp.zeros((2, 6), jnp.int32)

    def two16(r0, r1):
        return jnp.concatenate(
            [jnp.concatenate([r0, r1], axis=0), zpad], axis=1)

    neg1_ref[0] = two16(cat[0], cat[1])
    neg2_ref[0] = two16(cat[2], cat[3])
    widx1_ref[0] = two16(a1i0 * _N + cat[0], a1i1 * _N + cat[1])
    widx2_ref[0] = two16(cat[2] * _N + a2i0, cat[3] * _N + a2i1)
    dm_ref[0] = jnp.concatenate([jnp.zeros((1, 16), jnp.float32) + dm0,
                                 jnp.zeros((1, 16), jnp.float32) + dm1],
                                axis=0)


def _mine_grid_spec():
    full = lambda i, a1, a2: (0, 0)
    return pltpu.PrefetchScalarGridSpec(
        num_scalar_prefetch=2,
        grid=(_NA // 2,),
        in_specs=[
            pl.BlockSpec((_D, _NPAD), full),
            pl.BlockSpec((_D, _NPAD), full),
            pl.BlockSpec((1, 1, _D), lambda i, a1, a2: (a1[2 * i], 0, 0)),
            pl.BlockSpec((1, 1, _D), lambda i, a1, a2: (a2[2 * i], 0, 0)),
            pl.BlockSpec((1, 1, _D), lambda i, a1, a2: (a1[2 * i + 1], 0, 0)),
            pl.BlockSpec((1, 1, _D), lambda i, a1, a2: (a2[2 * i + 1], 0, 0)),
            pl.BlockSpec((_D, _D), full),
        ],
        out_specs=[
            pl.BlockSpec((1, 2, 16), lambda i, a1, a2: (i, 0, 0))
            for _ in range(5)
        ],
    )


def _mine_out_shapes():
    i32 = jnp.int32
    return [
        jax.ShapeDtypeStruct((_NA // 2, 2, 16), i32),
        jax.ShapeDtypeStruct((_NA // 2, 2, 16), i32),
        jax.ShapeDtypeStruct((_NA // 2, 2, 16), i32),
        jax.ShapeDtypeStruct((_NA // 2, 2, 16), i32),
        jax.ShapeDtypeStruct((_NA // 2, 2, 16), jnp.float32),
    ]


def _loss_kernel(o1_hbm, o2_hbm, otf_hbm, nidx1_hbm, nidx2_hbm,
                 widx1_hbm, widx2_hbm, aidx1_hbm, aidx2_hbm, dmp_hbm,
                 out_hbm,
                 idx_v, ae_v, ne_v, w_v, dmp_v, res_v, sem):
    c = lax.axis_index("c")
    s = lax.axis_index("s")
    wid = s * 2 + c
    base = wid * _P

    pltpu.sync_copy(dmp_hbm.at[pl.ds(base, _P)], dmp_v.at[pl.ds(0, _P)])

    total = jnp.float32(0.0)
    dirs = [
        (aidx1_hbm, nidx1_hbm, widx1_hbm, o1_hbm, o2_hbm),
        (aidx2_hbm, nidx2_hbm, widx2_hbm, o2_hbm, o1_hbm),
    ]
    for aidx_hbm, nidx_hbm, widx_hbm, bank_a, bank_n in dirs:
        for j in range(_P // _CH):
            off = base + j * _CH
            pltpu.sync_copy(aidx_hbm.at[pl.ds(off, _CH)], idx_v)
            pltpu.async_copy(bank_a.at[idx_v],
                             ae_v.at[pl.ds(j * _CH, _CH), :], sem).wait()
            pltpu.sync_copy(nidx_hbm.at[pl.ds(off, _CH)], idx_v)
            pltpu.async_copy(bank_n.at[idx_v],
                             ne_v.at[pl.ds(j * _CH, _CH), :], sem).wait()
            pltpu.sync_copy(widx_hbm.at[pl.ds(off, _CH)], idx_v)
            pltpu.async_copy(otf_hbm.at[idx_v],
                             w_v.at[pl.ds(j * _CH, _CH)], sem).wait()

        def body(p, acc):
            w = w_v[pl.ds(p, 16)][0]
            dm = dmp_v[pl.ds(p, 16)][0]
            s16 = jnp.zeros((16,), jnp.float32)
            for ch in range(_D // 16):
                a = ae_v[p, pl.ds(ch * 16, 16)]
                n = ne_v[p, pl.ds(ch * 16, 16)]
                s16 = s16 + jnp.abs(a - w * n)
            sdist = s16[0]
            for l in range(1, 16):
                sdist = sdist + s16[l]
            return acc + jnp.maximum(dm - sdist, 0.0)

        total = lax.fori_loop(0, _P, body, total)

    res_v[...] = jnp.where(lax.iota(jnp.int32, 16) == 0, total, 0.0)
    pltpu.sync_copy(res_v, out_hbm.at[wid])


def _make_loss_call():
    mesh = plsc.VectorSubcoreMesh(core_axis_name="c", subcore_axis_name="s")
    return functools.partial(
        pl.kernel,
        mesh=mesh,
        out_type=jax.ShapeDtypeStruct((_NW, 16), jnp.float32),
        scratch_types=[
            pltpu.VMEM((_CH,), jnp.int32),
            pltpu.VMEM((_P, _D), jnp.float32),
            pltpu.VMEM((_P, _D), jnp.float32),
            pltpu.VMEM((_P + 16,), jnp.float32),
            pltpu.VMEM((_P + 16,), jnp.float32),
            pltpu.VMEM((16,), jnp.float32),
            pltpu.SemaphoreType.DMA,
        ],
    )(_loss_kernel)


def kernel(out1, out2, anchor1, anchor2, ot_cost):
    a1 = anchor1.astype(jnp.int32)
    a2 = anchor2.astype(jnp.int32)
    pad = _NPAD - _N
    bt1 = jnp.pad(out1.T, ((0, 0), (0, pad)), constant_values=1e9)
    bt2 = jnp.pad(out2.T, ((0, 0), (0, pad)), constant_values=1e9)
    r1 = out1.reshape(_N, 1, _D)
    r2 = out2.reshape(_N, 1, _D)
    eye = jnp.eye(_D, dtype=jnp.float32)

    neg1o, neg2o, widx1o, widx2o, dmo = pl.pallas_call(
        _mine_kernel,
        grid_spec=_mine_grid_spec(),
        out_shape=_mine_out_shapes(),
    )(a1, a2, bt1, bt2, r1, r2, r1, r2, eye)

    nidx1 = neg1o[:, :, :_K].reshape(-1)
    nidx2 = neg2o[:, :, :_K].reshape(-1)
    widx1 = widx1o[:, :, :_K].reshape(-1)
    widx2 = widx2o[:, :, :_K].reshape(-1)
    dmp = dmo[:, :, :_K].reshape(-1)
    aidx1 = jnp.repeat(a1, _K)
    aidx2 = jnp.repeat(a2, _K)
    otf = ot_cost.reshape(-1)

    partial = _make_loss_call()(out1, out2, otf, nidx1, nidx2,
                                widx1, widx2, aidx1, aidx2, dmp)
    return jnp.sum(partial) / (_NA * _K)
